# baseline (speedup 1.0000x reference)
import functools

import jax
import jax.numpy as jnp
from jax import lax
from jax.experimental import pallas as pl
from jax.experimental.pallas import tpu as pltpu
from jax.experimental.pallas import tpu_sc as plsc

N = 10000
NP = 10240
E = 320000
F = 128
NC = 2
NS = 16
NW = NC * NS
EW_PER = E // NW
K = 80
CHUNKS = EW_PER // K

_mesh = plsc.VectorSubcoreMesh(
    core_axis_name="c", subcore_axis_name="s", num_cores=NC, num_subcores=NS)


KD = 128
NBT = E // KD // NW


@functools.partial(
    pl.kernel,
    out_type=jax.ShapeDtypeStruct((NC, NP), jnp.float32),
    mesh=_mesh,
    scratch_types=(
        [pltpu.VMEM((2, KD), jnp.int32)]
        + [pltpu.VMEM((KD,), jnp.float32)]
        + [pltpu.VMEM((NP // NS,), jnp.float32)]
        + [pltpu.VMEM_SHARED((NP,), jnp.float32)]
    ),
)
def _sc_degree(e_hbm, w_hbm, out_hbm, dbuf, wbuf, zbuf, dacc):
    c = lax.axis_index("c")
    s = lax.axis_index("s")
    wid = c * NS + s
    b0 = wid * NBT

    rows_per = NP // NS

    def zero_body(i, _):
        zbuf[pl.ds(i * 16, 16)] = jnp.zeros((16,), jnp.float32)
        return 0
    lax.fori_loop(0, rows_per // 16, zero_body, 0)
    pltpu.sync_copy(zbuf, dacc.at[pl.ds(s * rows_per, rows_per)])
    plsc.subcore_barrier()

    def block(i, _):
        off = (b0 + i) * KD
        pltpu.sync_copy(e_hbm.at[:, pl.ds(off, KD)], dbuf)
        pltpu.sync_copy(w_hbm.at[pl.ds(off, KD)], wbuf)
        pltpu.sync_copy(wbuf, dacc.at[dbuf.at[1]], add=True)
        return 0
    lax.fori_loop(0, NBT, block, 0)

    @pl.when(wid < E // KD - NBT * NW)
    def _extra():
        blk = NBT * NW + wid
        pltpu.sync_copy(e_hbm.at[:, pl.ds(blk * KD, KD)], dbuf)
        pltpu.sync_copy(w_hbm.at[pl.ds(blk * KD, KD)], wbuf)
        pltpu.sync_copy(wbuf, dacc.at[dbuf.at[1]], add=True)

    plsc.subcore_barrier()

    pltpu.sync_copy(dacc.at[pl.ds(s * rows_per, rows_per)],
                    out_hbm.at[c, pl.ds(s * rows_per, rows_per)])


AP = 10112
_RPT = AP // NS


@functools.partial(
    pl.kernel,
    out_type=jax.ShapeDtypeStruct((NC, AP, F), jnp.float32),
    mesh=_mesh,
    scratch_types=(
        [pltpu.VMEM((2, K), jnp.int32) for _ in range(4)]
        + [pltpu.VMEM((1, K), jnp.float32) for _ in range(4)]
        + [pltpu.VMEM((K, F), jnp.float32) for _ in range(4)]
        + [pltpu.VMEM_SHARED((AP, F), jnp.float32)]
        + [pltpu.SemaphoreType.DMA for _ in range(12)]
    ),
)
def _sc_agg(h_hbm, e_hbm, w_hbm, out_hbm,
            eb0, eb1, eb2, eb3, wb0, wb1, wb2, wb3, r0, r1, r2, r3, acc,
            se0, se1, se2, se3, sg0, sg1, sg2, sg3, ss0, ss1, ss2, ss3):
    c = lax.axis_index("c")
    s = lax.axis_index("s")
    wid = c * NS + s
    ebs = [eb0, eb1, eb2, eb3]
    wbs = [wb0, wb1, wb2, wb3]
    rs = [r0, r1, r2, r3]
    ses = [se0, se1, se2, se3]
    sgs = [sg0, sg1, sg2, sg3]
    sss = [ss0, ss1, ss2, ss3]

    def issue_edge(ch, t):
        pltpu.async_copy(e_hbm.at[0, wid, ch, 0], ebs[t].at[0], ses[t])
        pltpu.async_copy(e_hbm.at[1, wid, ch, 0], ebs[t].at[1], ses[t])
        pltpu.async_copy(w_hbm.at[wid, ch, 0], wbs[t].at[0], ses[t])

    def wait_edge(t):
        pltpu.make_async_copy(e_hbm.at[0, wid, 0, 0],
                              ebs[t].at[0], ses[t]).wait()
        pltpu.make_async_copy(e_hbm.at[1, wid, 0, 0],
                              ebs[t].at[1], ses[t]).wait()
        pltpu.make_async_copy(w_hbm.at[wid, 0, 0],
                              wbs[t].at[0], ses[t]).wait()

    def issue_gather(t):
        pltpu.async_copy(h_hbm.at[ebs[t].at[0]], rs[t], sgs[t])

    def wait_gather(t):
        pltpu.make_async_copy(h_hbm.at[ebs[t].at[0]], rs[t], sgs[t]).wait()

    def issue_scatter(t):
        pltpu.async_copy(rs[t], acc.at[ebs[t].at[1]], sss[t], add=True)

    def wait_scatter(t):
        pltpu.make_async_copy(rs[t], acc.at[ebs[t].at[1]], sss[t]).wait()

    def mult(t):
        def grp_body(g, _):
            w16 = wbs[t][0, pl.ds(g * 16, 16)]
            for u in range(16):
                r = g * 16 + u
                w = w16[u]
                for f in range(F // 16):
                    rs[t][r, pl.ds(f * 16, 16)] = (
                        rs[t][r, pl.ds(f * 16, 16)] * w)
            return 0
        lax.fori_loop(0, K // 16, grp_body, 0)

    def chunk(ch, t, wait_sc=True, edge_next=True):
        wait_edge((t + 1) % 4)
        if wait_sc:
            wait_scatter((t + 2) % 4)
        issue_gather((t + 1) % 4)
        if edge_next:
            issue_edge(ch + 2, (t + 2) % 4)
        wait_gather(t)
        mult(t)
        issue_scatter(t)

    def zero_body(i, _):
        for f in range(F // 16):
            r0[i, pl.ds(f * 16, 16)] = jnp.zeros((16,), jnp.float32)
        return 0
    lax.fori_loop(0, K, zero_body, 0)
    for z in range(7):
        pltpu.sync_copy(r0, acc.at[pl.ds(s * _RPT + z * K, K)])
    pltpu.sync_copy(r0.at[pl.ds(0, _RPT - 7 * K)],
                    acc.at[pl.ds(s * _RPT + 7 * K, _RPT - 7 * K)])
    plsc.subcore_barrier()

    issue_edge(0, 0)
    issue_edge(1, 1)
    wait_edge(0)
    issue_gather(0)

    chunk(0, 0, wait_sc=False)
    chunk(1, 1, wait_sc=False)
    chunk(2, 2)

    def steady(jo, _):
        base = 3 + jo * 4
        for b in range(4):
            chunk(base + b, (3 + b) % 4)
        return 0
    lax.fori_loop(0, (CHUNKS - 5) // 4, steady, 0)

    chunk(CHUNKS - 2, (CHUNKS - 2) % 4, edge_next=False)
    t_last = (CHUNKS - 1) % 4
    wait_scatter((t_last + 2) % 4)
    wait_gather(t_last)
    mult(t_last)
    issue_scatter(t_last)
    wait_scatter((t_last + 3) % 4)
    wait_scatter(t_last)
    plsc.subcore_barrier()

    pltpu.sync_copy(acc.at[pl.ds(s * _RPT, _RPT)],
                    out_hbm.at[c, pl.ds(s * _RPT, _RPT)])


_RB = 1000


def _tc1_body(deg_ref, x_ref, w_ref, h_ref, dis_ref):
    deg = deg_ref[0] + deg_ref[1] + 1.0
    dis = jnp.where(deg > 0, lax.rsqrt(deg), 0.0)
    dis_ref[...] = dis
    h_ref[...] = dis * jnp.dot(x_ref[...], w_ref[...],
                               preferred_element_type=jnp.float32)


def _tc_lin1(deg_parts, x, W1):
    return pl.pallas_call(
        _tc1_body,
        grid=(N // _RB,),
        in_specs=[
            pl.BlockSpec((2, _RB, 1), lambda i: (0, i, 0)),
            pl.BlockSpec((_RB, F), lambda i: (i, 0)),
            pl.BlockSpec((F, F), lambda i: (0, 0)),
        ],
        out_specs=[
            pl.BlockSpec((_RB, F), lambda i: (i, 0)),
            pl.BlockSpec((_RB, 1), lambda i: (i, 0)),
        ],
        out_shape=[
            jax.ShapeDtypeStruct((N, F), jnp.float32),
            jax.ShapeDtypeStruct((N, 1), jnp.float32),
        ],
    )(deg_parts, x, W1)


def _tc2_body(p_ref, hp_ref, dis_ref, b_ref, w_ref, out_ref):
    dis = dis_ref[...]
    t = dis * (p_ref[0] + p_ref[1] + hp_ref[...]) + b_ref[...]
    t = jnp.maximum(t, 0.0)
    out_ref[...] = dis * jnp.dot(t, w_ref[...],
                                 preferred_element_type=jnp.float32)


def _tc_mid(parts, hp, dis, b1, W2):
    return pl.pallas_call(
        _tc2_body,
        grid=(N // _RB,),
        in_specs=[
            pl.BlockSpec((2, _RB, F), lambda i: (0, i, 0)),
            pl.BlockSpec((_RB, F), lambda i: (i, 0)),
            pl.BlockSpec((_RB, 1), lambda i: (i, 0)),
            pl.BlockSpec((1, F), lambda i: (0, 0)),
            pl.BlockSpec((F, F), lambda i: (0, 0)),
        ],
        out_specs=pl.BlockSpec((_RB, F), lambda i: (i, 0)),
        out_shape=jax.ShapeDtypeStruct((N, F), jnp.float32),
    )(parts, hp, dis, b1, W2)


def _tc3_body(p_ref, hp_ref, dis_ref, b_ref, out_ref):
    t = dis_ref[...] * (p_ref[0] + p_ref[1] + hp_ref[...]) + b_ref[...]
    out_ref[...] = jnp.maximum(t, 0.0)


def _tc_out(parts, hp, dis, b2):
    return pl.pallas_call(
        _tc3_body,
        grid=(N // _RB,),
        in_specs=[
            pl.BlockSpec((2, _RB, F), lambda i: (0, i, 0)),
            pl.BlockSpec((_RB, F), lambda i: (i, 0)),
            pl.BlockSpec((_RB, 1), lambda i: (i, 0)),
            pl.BlockSpec((1, F), lambda i: (0, 0)),
        ],
        out_specs=pl.BlockSpec((_RB, F), lambda i: (i, 0)),
        out_shape=jax.ShapeDtypeStruct((N, F), jnp.float32),
    )(parts, hp, dis, b2)


@jax.jit
def kernel(x, edge_index, edge_attr, W1, b1, W2, b2):
    ei5 = edge_index.reshape(2, NW, CHUNKS, 1, K)
    ew4 = edge_attr.reshape(NW, CHUNKS, 1, K)

    deg_parts = _sc_degree(edge_index, edge_attr)
    deg_parts = deg_parts[:, :N].reshape(2, N, 1)

    h1, dis = _tc_lin1(deg_parts, x, W1)
    agg1 = _sc_agg(h1, ei5, ew4)
    h2 = _tc_mid(agg1, h1, dis, b1.reshape(1, F), W2)
    agg2 = _sc_agg(h2, ei5, ew4)
    out = _tc_out(agg2, h2, dis, b2.reshape(1, F))
    return out.reshape(-1)


# speedup vs baseline: 1.1949x; 1.1949x over previous
import functools

import jax
import jax.numpy as jnp
from jax import lax
from jax.experimental import pallas as pl
from jax.experimental.pallas import tpu as pltpu
from jax.experimental.pallas import tpu_sc as plsc

N = 10000
NP = 10240
E = 320000
F = 128
NC = 2
NS = 16
NW = NC * NS
EW_PER = E // NW
K = 80
CHUNKS = EW_PER // K

_mesh = plsc.VectorSubcoreMesh(
    core_axis_name="c", subcore_axis_name="s", num_cores=NC, num_subcores=NS)


KD = 128
NBT = E // KD // NW


@functools.partial(
    pl.kernel,
    out_type=jax.ShapeDtypeStruct((NC, NP), jnp.float32),
    mesh=_mesh,
    scratch_types=(
        [pltpu.VMEM((2, KD), jnp.int32) for _ in range(4)]
        + [pltpu.VMEM((KD,), jnp.float32) for _ in range(4)]
        + [pltpu.VMEM((NP // NS,), jnp.float32)]
        + [pltpu.VMEM_SHARED((NP,), jnp.float32)]
        + [pltpu.SemaphoreType.DMA for _ in range(4)]
    ),
)
def _sc_degree(e_hbm, w_hbm, out_hbm, dr0, dr1, dr2, dr3,
               wr0, wr1, wr2, wr3, zbuf, dacc, sd0, sd1, sd2, sd3):
    c = lax.axis_index("c")
    s = lax.axis_index("s")
    wid = c * NS + s
    b0 = wid * NBT
    drs = [dr0, dr1, dr2, dr3]
    wrs = [wr0, wr1, wr2, wr3]
    sds = [sd0, sd1, sd2, sd3]

    def issue(blk, t):
        off = blk * KD
        pltpu.async_copy(e_hbm.at[:, pl.ds(off, KD)], drs[t], sds[t])
        pltpu.async_copy(w_hbm.at[pl.ds(off, KD)], wrs[t], sds[t])

    def wait(t):
        pltpu.make_async_copy(e_hbm.at[:, pl.ds(0, KD)], drs[t], sds[t]).wait()
        pltpu.make_async_copy(w_hbm.at[pl.ds(0, KD)], wrs[t], sds[t]).wait()

    rows_per = NP // NS

    def zero_body(i, _):
        zbuf[pl.ds(i * 16, 16)] = jnp.zeros((16,), jnp.float32)
        return 0
    lax.fori_loop(0, rows_per // 16, zero_body, 0)
    pltpu.sync_copy(zbuf, dacc.at[pl.ds(s * rows_per, rows_per)])
    plsc.subcore_barrier()

    for t in range(3):
        issue(b0 + t, t)

    def chunk(i, t):
        wait(t)
        pltpu.sync_copy(wrs[t], dacc.at[drs[t].at[1]], add=True)
        issue(b0 + jnp.minimum(i + 3, NBT - 1), (t + 3) % 4)

    chunk(0, 0)
    chunk(1, 1)

    def steady(jo, _):
        for b in range(4):
            chunk(2 + jo * 4 + b, (2 + b) % 4)
        return 0
    lax.fori_loop(0, (NBT - 2) // 4, steady, 0)
    for t in (NBT % 4, (NBT + 1) % 4, (NBT + 2) % 4):
        wait(t)

    @pl.when(wid < E // KD - NBT * NW)
    def _extra():
        blk = NBT * NW + wid
        pltpu.sync_copy(e_hbm.at[:, pl.ds(blk * KD, KD)], drs[0])
        pltpu.sync_copy(w_hbm.at[pl.ds(blk * KD, KD)], wrs[0])
        pltpu.sync_copy(wrs[0], dacc.at[drs[0].at[1]], add=True)

    plsc.subcore_barrier()

    pltpu.sync_copy(dacc.at[pl.ds(s * rows_per, rows_per)],
                    out_hbm.at[c, pl.ds(s * rows_per, rows_per)])


AP = 10112
_RPT = AP // NS


@functools.partial(
    pl.kernel,
    out_type=jax.ShapeDtypeStruct((NC, AP, F), jnp.float32),
    mesh=_mesh,
    scratch_types=(
        [pltpu.VMEM((2, K), jnp.int32) for _ in range(4)]
        + [pltpu.VMEM((1, K), jnp.float32) for _ in range(4)]
        + [pltpu.VMEM((K, F), jnp.float32) for _ in range(4)]
        + [pltpu.VMEM_SHARED((AP, F), jnp.float32)]
        + [pltpu.SemaphoreType.DMA for _ in range(12)]
    ),
)
def _sc_agg(h_hbm, e_hbm, w_hbm, out_hbm,
            eb0, eb1, eb2, eb3, wb0, wb1, wb2, wb3, r0, r1, r2, r3, acc,
            se0, se1, se2, se3, sg0, sg1, sg2, sg3, ss0, ss1, ss2, ss3):
    c = lax.axis_index("c")
    s = lax.axis_index("s")
    wid = c * NS + s
    ebs = [eb0, eb1, eb2, eb3]
    wbs = [wb0, wb1, wb2, wb3]
    rs = [r0, r1, r2, r3]
    ses = [se0, se1, se2, se3]
    sgs = [sg0, sg1, sg2, sg3]
    sss = [ss0, ss1, ss2, ss3]

    def issue_edge(ch, t):
        pltpu.async_copy(e_hbm.at[0, wid, ch, 0], ebs[t].at[0], ses[t])
        pltpu.async_copy(e_hbm.at[1, wid, ch, 0], ebs[t].at[1], ses[t])
        pltpu.async_copy(w_hbm.at[wid, ch, 0], wbs[t].at[0], ses[t])

    def wait_edge(t):
        pltpu.make_async_copy(e_hbm.at[0, wid, 0, 0],
                              ebs[t].at[0], ses[t]).wait()
        pltpu.make_async_copy(e_hbm.at[1, wid, 0, 0],
                              ebs[t].at[1], ses[t]).wait()
        pltpu.make_async_copy(w_hbm.at[wid, 0, 0],
                              wbs[t].at[0], ses[t]).wait()

    def issue_gather(t):
        pltpu.async_copy(h_hbm.at[ebs[t].at[0]], rs[t], sgs[t])

    def wait_gather(t):
        pltpu.make_async_copy(h_hbm.at[ebs[t].at[0]], rs[t], sgs[t]).wait()

    def issue_scatter(t):
        pltpu.async_copy(rs[t], acc.at[ebs[t].at[1]], sss[t], add=True)

    def wait_scatter(t):
        pltpu.make_async_copy(rs[t], acc.at[ebs[t].at[1]], sss[t]).wait()

    def mult(t):
        def grp_body(g, _):
            w16 = wbs[t][0, pl.ds(g * 16, 16)]
            for u in range(16):
                r = g * 16 + u
                w = w16[u]
                for f in range(F // 16):
                    rs[t][r, pl.ds(f * 16, 16)] = (
                        rs[t][r, pl.ds(f * 16, 16)] * w)
            return 0
        lax.fori_loop(0, K // 16, grp_body, 0)

    def chunk(ch, t, wait_sc=True, edge_next=True):
        wait_edge((t + 1) % 4)
        if wait_sc:
            wait_scatter((t + 2) % 4)
        issue_gather((t + 1) % 4)
        if edge_next:
            issue_edge(ch + 2, (t + 2) % 4)
        wait_gather(t)
        mult(t)
        issue_scatter(t)

    def zero_body(i, _):
        for f in range(F // 16):
            r0[i, pl.ds(f * 16, 16)] = jnp.zeros((16,), jnp.float32)
        return 0
    lax.fori_loop(0, K, zero_body, 0)
    for z in range(7):
        pltpu.sync_copy(r0, acc.at[pl.ds(s * _RPT + z * K, K)])
    pltpu.sync_copy(r0.at[pl.ds(0, _RPT - 7 * K)],
                    acc.at[pl.ds(s * _RPT + 7 * K, _RPT - 7 * K)])
    plsc.subcore_barrier()

    issue_edge(0, 0)
    issue_edge(1, 1)
    wait_edge(0)
    issue_gather(0)

    chunk(0, 0, wait_sc=False)
    chunk(1, 1, wait_sc=False)
    chunk(2, 2)

    def steady(jo, _):
        base = 3 + jo * 4
        for b in range(4):
            chunk(base + b, (3 + b) % 4)
        return 0
    lax.fori_loop(0, (CHUNKS - 5) // 4, steady, 0)

    chunk(CHUNKS - 2, (CHUNKS - 2) % 4, edge_next=False)
    t_last = (CHUNKS - 1) % 4
    wait_scatter((t_last + 2) % 4)
    wait_gather(t_last)
    mult(t_last)
    issue_scatter(t_last)
    wait_scatter((t_last + 3) % 4)
    wait_scatter(t_last)
    plsc.subcore_barrier()

    pltpu.sync_copy(acc.at[pl.ds(s * _RPT, _RPT)],
                    out_hbm.at[c, pl.ds(s * _RPT, _RPT)])


_RB = 1000


def _tc1_body(deg_ref, x_ref, w_ref, h_ref, dis_ref):
    deg = deg_ref[0] + deg_ref[1] + 1.0
    dis = jnp.where(deg > 0, lax.rsqrt(deg), 0.0)
    dis_ref[...] = dis
    h_ref[...] = dis * jnp.dot(x_ref[...], w_ref[...],
                               preferred_element_type=jnp.float32)


def _tc_lin1(deg_parts, x, W1):
    return pl.pallas_call(
        _tc1_body,
        grid=(N // _RB,),
        in_specs=[
            pl.BlockSpec((2, _RB, 1), lambda i: (0, i, 0)),
            pl.BlockSpec((_RB, F), lambda i: (i, 0)),
            pl.BlockSpec((F, F), lambda i: (0, 0)),
        ],
        out_specs=[
            pl.BlockSpec((_RB, F), lambda i: (i, 0)),
            pl.BlockSpec((_RB, 1), lambda i: (i, 0)),
        ],
        out_shape=[
            jax.ShapeDtypeStruct((N, F), jnp.float32),
            jax.ShapeDtypeStruct((N, 1), jnp.float32),
        ],
    )(deg_parts, x, W1)


def _tc2_body(p_ref, hp_ref, dis_ref, b_ref, w_ref, out_ref):
    dis = dis_ref[...]
    t = dis * (p_ref[0] + p_ref[1] + hp_ref[...]) + b_ref[...]
    t = jnp.maximum(t, 0.0)
    out_ref[...] = dis * jnp.dot(t, w_ref[...],
                                 preferred_element_type=jnp.float32)


def _tc_mid(parts, hp, dis, b1, W2):
    return pl.pallas_call(
        _tc2_body,
        grid=(N // _RB,),
        in_specs=[
            pl.BlockSpec((2, _RB, F), lambda i: (0, i, 0)),
            pl.BlockSpec((_RB, F), lambda i: (i, 0)),
            pl.BlockSpec((_RB, 1), lambda i: (i, 0)),
            pl.BlockSpec((1, F), lambda i: (0, 0)),
            pl.BlockSpec((F, F), lambda i: (0, 0)),
        ],
        out_specs=pl.BlockSpec((_RB, F), lambda i: (i, 0)),
        out_shape=jax.ShapeDtypeStruct((N, F), jnp.float32),
    )(parts, hp, dis, b1, W2)


def _tc3_body(p_ref, hp_ref, dis_ref, b_ref, out_ref):
    t = dis_ref[...] * (p_ref[0] + p_ref[1] + hp_ref[...]) + b_ref[...]
    out_ref[...] = jnp.maximum(t, 0.0)


def _tc_out(parts, hp, dis, b2):
    return pl.pallas_call(
        _tc3_body,
        grid=(N // _RB,),
        in_specs=[
            pl.BlockSpec((2, _RB, F), lambda i: (0, i, 0)),
            pl.BlockSpec((_RB, F), lambda i: (i, 0)),
            pl.BlockSpec((_RB, 1), lambda i: (i, 0)),
            pl.BlockSpec((1, F), lambda i: (0, 0)),
        ],
        out_specs=pl.BlockSpec((_RB, F), lambda i: (i, 0)),
        out_shape=jax.ShapeDtypeStruct((N, F), jnp.float32),
    )(parts, hp, dis, b2)


@jax.jit
def kernel(x, edge_index, edge_attr, W1, b1, W2, b2):
    ei5 = edge_index.reshape(2, NW, CHUNKS, 1, K)
    ew4 = edge_attr.reshape(NW, CHUNKS, 1, K)

    deg_parts = _sc_degree(edge_index, edge_attr)
    deg_parts = deg_parts[:, :N].reshape(2, N, 1)

    h1, dis = _tc_lin1(deg_parts, x, W1)
    agg1 = _sc_agg(h1, ei5, ew4)
    h2 = _tc_mid(agg1, h1, dis, b1.reshape(1, F), W2)
    agg2 = _sc_agg(h2, ei5, ew4)
    out = _tc_out(agg2, h2, dis, b2.reshape(1, F))
    return out.reshape(-1)
